# baseline (speedup 1.0000x reference)
import jax
import jax.numpy as jnp
import numpy as np
from jax import lax
from jax.experimental import pallas as pl
from jax.experimental.pallas import tpu as pltpu
from jax.experimental.pallas import tpu_sc as plsc

N = 5000
NPAD = 5120
NW = 16
OWN = NPAD // NW
OWNC = OWN // 16
LPAD = OWN + 16
PUB = NW * 32
NEG = np.float32(-1e30)
NEG_HALF = np.float32(-5e29)
BLO = np.float32(-3e38)
SCORE_T = np.float32(0.2)
IOU_T = np.float32(0.5)
HUMAN = np.int32(1)
MAXH = np.int32(15)
MAXO = np.int32(15)
NOUT = 30

_GDN = lax.GatherDimensionNumbers(
    offset_dims=(), collapsed_slice_dims=(0,), start_index_map=(0,))


def _perm(x, idx):
    return lax.gather(x, idx[:, None], _GDN, (1,),
                      mode=lax.GatherScatterMode.PROMISE_IN_BOUNDS)


def _bt(am, ai, bm, bi):
    return (am > bm) | ((am == bm) & (ai < bi))


def _argmax2_splat(m1, i1, m2, i2, lanes):
    for k in (8, 4, 2, 1):
        pidx = lanes ^ k
        pm1 = _perm(m1, pidx)
        pi1 = _perm(i1, pidx)
        pm2 = _perm(m2, pidx)
        pi2 = _perm(i2, pidx)
        b1 = _bt(pm1, pi1, m1, i1)
        n1m = jnp.where(b1, pm1, m1)
        n1i = jnp.where(b1, pi1, i1)
        lm = jnp.where(b1, m1, pm1)
        li = jnp.where(b1, i1, pi1)
        b2 = _bt(m2, i2, pm2, pi2)
        c2m = jnp.where(b2, m2, pm2)
        c2i = jnp.where(b2, i2, pi2)
        b3 = _bt(lm, li, c2m, c2i)
        m1, i1 = n1m, n1i
        m2 = jnp.where(b3, lm, c2m)
        i2 = jnp.where(b3, li, c2i)
    return m1, i1, m2, i2


def _sc_nms(x1h, y1h, x2h, y2h, sh, labh, pack_o, olab_o, oval_o,
            lx1, ly1, lx2, ly2, ls, llab,
            lox1, loy1, lox2, loy2, lkey, lfill,
            pack_v, olab_v, oval_v, rowbuf, fbrow, hno, puball, pub_sh):
    cid = lax.axis_index("c")
    sid = lax.axis_index("s")
    lanes = lax.iota(jnp.int32, 16)
    base = sid * OWN
    is_out = (cid == 0) & (sid == 0)

    pltpu.sync_copy(x1h.at[pl.ds(base, OWN)], lx1.at[pl.ds(0, OWN)])
    pltpu.sync_copy(y1h.at[pl.ds(base, OWN)], ly1.at[pl.ds(0, OWN)])
    pltpu.sync_copy(x2h.at[pl.ds(base, OWN)], lx2.at[pl.ds(0, OWN)])
    pltpu.sync_copy(y2h.at[pl.ds(base, OWN)], ly2.at[pl.ds(0, OWN)])
    pltpu.sync_copy(sh.at[pl.ds(base, OWN)], ls.at[pl.ds(0, OWN)])
    pltpu.sync_copy(labh.at[pl.ds(base, OWN)], llab.at[pl.ds(0, OWN)])

    def publish_round(off):
        pltpu.sync_copy(rowbuf.at[pl.ds(0, 16)],
                        pub_sh.at[pl.ds(off + sid * 32, 16)])
        pltpu.sync_copy(rowbuf.at[pl.ds(16, 16)],
                        pub_sh.at[pl.ds(off + sid * 32 + 16, 16)])
        plsc.subcore_barrier()
        pltpu.sync_copy(pub_sh.at[pl.ds(off, PUB)], puball)

    def reduce2_rows():
        bm1 = BLO
        bi1 = np.float32(3e38)
        bm2 = BLO
        bi2 = np.float32(3e38)
        brow1 = jnp.zeros((16,), jnp.float32)
        brow2 = jnp.zeros((16,), jnp.float32)
        for r in range(2 * NW):
            rrow = puball[pl.ds(r * 16, 16)]
            mr = rrow[0]
            ir = rrow[1]
            u = (mr > bm1) | ((mr == bm1) & (ir < bi1))
            v = (mr > bm2) | ((mr == bm2) & (ir < bi2))
            uf = jnp.where(u, np.float32(1.0), np.float32(0.0))
            vf = jnp.where(v, np.float32(1.0), np.float32(0.0))
            ufv = jnp.full((16,), uf, jnp.float32)
            vfv = jnp.full((16,), vf, jnp.float32)
            brow2 = (ufv * brow1
                     + (1.0 - ufv) * (vfv * rrow + (1.0 - vfv) * brow2))
            brow1 = ufv * rrow + (1.0 - ufv) * brow1
            bm2 = jnp.where(u, bm1, jnp.where(v, mr, bm2))
            bi2 = jnp.where(u, bi1, jnp.where(v, ir, bi2))
            bm1 = jnp.where(u, mr, bm1)
            bi1 = jnp.where(u, ir, bi1)
        return brow1, brow2

    def mk_row(m, mi, with_coords):
        ms = m[0]
        gi = mi[0]
        loff = jnp.clip(gi - base, 0, OWN - 1)
        fd = pl.ds(loff, 16)
        if with_coords:
            coords = (lox1[fd][0], loy1[fd][0], lox2[fd][0], loy2[fd][0])
        else:
            z = np.float32(0.0)
            coords = (z, z, z, z)
        vals = (ms, gi.astype(jnp.float32)) + coords + (
            lx1[fd][0], ly1[fd][0], lx2[fd][0], ly2[fd][0],
            ls[fd][0], llab[fd][0].astype(jnp.float32))
        row = jnp.full((16,), np.float32(0.0), jnp.float32)
        for j, v in enumerate(vals):
            row = jnp.where(lanes == j, v, row)
        return row

    negrow = jnp.where(lanes == 0, NEG, np.float32(0.0))

    def fill_kill(pick):
        loffp = jnp.clip(pick - base, 0, OWN - 1)
        inr = (pick >= base) & (pick < base + OWN)
        kd = pl.ds(loffp, 16)
        fold = lfill[kd]
        v0 = jnp.where(inr, NEG, fold[0])
        lfill[kd] = jnp.where(lanes == 0, v0, fold)

    def s1(c, m):
        d = pl.ds(c * 16, 16)
        return jnp.maximum(m, jnp.maximum(lx2[d], ly2[d]))
    mloc = lax.fori_loop(0, OWNC, s1, jnp.full((16,), NEG, jnp.float32))
    for k in (8, 4, 2, 1):
        mloc = jnp.maximum(mloc, _perm(mloc, lanes ^ k))
    rowbuf[pl.ds(0, 16)] = jnp.where(lanes == 0, mloc[0], np.float32(0.0))
    rowbuf[pl.ds(16, 16)] = negrow
    publish_round(2 * PUB)
    mc = BLO
    for r in range(NW):
        mc = jnp.maximum(mc, puball[pl.ds(r * 32, 16)][0])
    mc = mc + 1.0

    def s2(c, carry):
        m1, i1, m2, i2 = carry
        d = pl.ds(c * 16, 16)
        idxv = base + c * 16 + lanes
        off = llab[d].astype(jnp.float32) * mc
        a = lx1[d] + off
        b = ly1[d] + off
        cc = lx2[d] + off
        dd = ly2[d] + off
        lox1[d] = a
        loy1[d] = b
        lox2[d] = cc
        loy2[d] = dd
        sc = ls[d]
        vmask = sc >= SCORE_T
        keyc = jnp.where(vmask, sc, NEG)
        lkey[d] = keyc
        idxf = idxv.astype(jnp.float32)
        lfill[d] = jnp.where(vmask, sc,
                             jnp.where(idxv < N, -(idxf + 2.0), NEG))
        upd = keyc > m1
        nm1 = jnp.where(upd, keyc, m1)
        ni1 = jnp.where(upd, idxv, i1)
        dm = jnp.where(upd, m1, keyc)
        di = jnp.where(upd, i1, idxv)
        upd2 = dm > m2
        return (nm1, ni1, jnp.where(upd2, dm, m2), jnp.where(upd2, di, i2))

    zstate = (jnp.full((16,), NEG, jnp.float32), jnp.zeros((16,), jnp.int32),
              jnp.full((16,), NEG, jnp.float32), jnp.zeros((16,), jnp.int32))
    m1, i1, m2, i2 = lax.fori_loop(0, OWNC, s2, zstate)
    m1, i1, m2, i2 = _argmax2_splat(m1, i1, m2, i2, lanes)
    rowbuf[pl.ds(0, 16)] = mk_row(m1, i1, True)
    rowbuf[pl.ds(16, 16)] = mk_row(m2, i2, True)
    fbrow[...] = jnp.zeros((16,), jnp.float32)
    hno[0] = jnp.int32(0)
    hno[1] = jnp.int32(0)
    hno[2] = jnp.int32(0)
    publish_round(PUB)

    @pl.when(is_out)
    def _():
        zf = jnp.zeros((16,), jnp.float32)
        zi = jnp.zeros((16,), jnp.int32)
        for kk in range(11):
            pack_v[pl.ds(kk * 16, 16)] = zf
        for kk in range(3):
            olab_v[pl.ds(kk * 16, 16)] = zi
            oval_v[pl.ds(kk * 16, 16)] = zi

    def body(t, carry):
        h = hno[0]
        o = hno[1]
        ns = hno[2]

        @pl.when(ns < NOUT)
        def _():
            browA, browB = reduce2_rows()
            active = browA[0] > NEG_HALF

            @pl.when(jnp.logical_not(active))
            def _():
                def fsweep(c, fcarry):
                    m, mi = fcarry
                    fc = lfill[pl.ds(c * 16, 16)]
                    idxv = base + c * 16 + lanes
                    upd = fc > m
                    return jnp.where(upd, fc, m), jnp.where(upd, idxv, mi)
                fm, fmi = lax.fori_loop(
                    0, OWNC, fsweep,
                    (jnp.full((16,), NEG, jnp.float32),
                     jnp.zeros((16,), jnp.int32)))
                for k in (8, 4, 2, 1):
                    pidx = lanes ^ k
                    pfm = _perm(fm, pidx)
                    pfi = _perm(fmi, pidx)
                    bf = _bt(pfm, pfi, fm, fmi)
                    fm = jnp.where(bf, pfm, fm)
                    fmi = jnp.where(bf, pfi, fmi)
                rowbuf[pl.ds(0, 16)] = mk_row(fm, fmi, False)
                rowbuf[pl.ds(16, 16)] = negrow
                publish_round(2 * PUB)
                fa, _fb = reduce2_rows()
                fbrow[...] = fa

            af = jnp.where(active, np.float32(1.0), np.float32(0.0))
            afv = jnp.full((16,), af, jnp.float32)
            arow = afv * browA + (1.0 - afv) * fbrow[...]
            pick_a = arow[1].astype(jnp.int32)
            plab_a = arow[11].astype(jnp.int32)
            ish_a = plab_a == HUMAN
            inca = active.astype(jnp.int32)
            h2a = h + jnp.where(ish_a, inca, 0)
            o2a = o + jnp.where(ish_a, 0, inca)
            pick_b = browB[1].astype(jnp.int32)
            plab_b = browB[11].astype(jnp.int32)
            ish_b = plab_b == HUMAN
            bexist = browB[0] > NEG_HALF
            capok = jnp.where(ish_b, h2a < MAXH, o2a < MAXO)
            ax1 = arow[2]
            ay1 = arow[3]
            ax2 = arow[4]
            ay2 = arow[5]
            aarea = (ax2 - ax1) * (ay2 - ay1)
            bx1 = browB[2]
            by1 = browB[3]
            bx2 = browB[4]
            by2 = browB[5]
            barea = (bx2 - bx1) * (by2 - by1)
            ltx = jnp.maximum(ax1, bx1)
            lty = jnp.maximum(ay1, by1)
            rbx = jnp.minimum(ax2, bx2)
            rby = jnp.minimum(ay2, by2)
            wab = jnp.maximum(rbx - ltx, 0.0)
            hab = jnp.maximum(rby - lty, 0.0)
            iab = wab * hab
            uab = aarea + barea - iab
            iabv = jnp.full((16,), iab, jnp.float32)
            uabv = jnp.full((16,), jnp.maximum(uab, np.float32(1e-8)),
                            jnp.float32)
            supv = jnp.where((iabv / uabv) > IOU_T,
                             np.float32(1.0), np.float32(0.0))
            rowbuf[pl.ds(0, 16)] = supv
            supab = rowbuf[pl.ds(0, 16)][0] > np.float32(0.5)
            bsel = (active & bexist & capok
                    & jnp.logical_not(supab) & (ns <= NOUT - 2))
            incb = bsel.astype(jnp.int32)
            h2 = h2a + jnp.where(ish_b, incb, 0)
            o2 = o2a + jnp.where(ish_b, 0, incb)

            fill_kill(pick_a)
            fill_kill(jnp.where(bsel, pick_b, jnp.int32(-1)))

            @pl.when(is_out)
            def _():
                vals = jnp.where(lanes == 0, arow[6],
                       jnp.where(lanes == 1, arow[7],
                       jnp.where(lanes == 2, arow[8],
                       jnp.where(lanes == 3, arow[9], arow[10]))))
                pd = pl.ds(ns * 5, 16)
                pold = pack_v[pd]
                pack_v[pd] = jnp.where(lanes < 5, vals, pold)
                od = pl.ds(ns, 16)
                lold = olab_v[od]
                olab_v[od] = jnp.where(lanes == 0, plab_a, lold)
                vold = oval_v[od]
                oval_v[od] = jnp.where(lanes == 0, inca, vold)

                bsf = jnp.full((16,), incb.astype(jnp.float32), jnp.float32)
                bvals = jnp.where(lanes == 0, browB[6],
                        jnp.where(lanes == 1, browB[7],
                        jnp.where(lanes == 2, browB[8],
                        jnp.where(lanes == 3, browB[9], browB[10]))))
                pdb = pl.ds((ns + 1) * 5, 16)
                poldb = pack_v[pdb]
                pack_v[pdb] = jnp.where(
                    lanes < 5, bsf * bvals + (1.0 - bsf) * poldb, poldb)
                odb = pl.ds(ns + 1, 16)
                loldb = olab_v[odb]
                olab_v[odb] = jnp.where(
                    lanes == 0, incb * plab_b + (1 - incb) * loldb[0], loldb)
                voldb = oval_v[odb]
                oval_v[odb] = jnp.where(
                    lanes == 0, incb + (1 - incb) * voldb[0], voldb)

            rowbuf[pl.ds(0, 16)] = negrow
            rowbuf[pl.ds(16, 16)] = negrow

            @pl.when(active)
            def _():
                bsf2 = jnp.where(bsel, np.float32(1.0), np.float32(0.0))
                sbx1 = bsf2 * bx1
                sby1 = bsf2 * by1
                sbx2 = bsf2 * bx2
                sby2 = bsf2 * by2
                sbarea = (sbx2 - sbx1) * (sby2 - sby1)

                caph = (h2 >= MAXH) & (h < MAXH)
                capo = (o2 >= MAXO) & (o < MAXO)

                @pl.when(caph)
                def _():
                    def ksweep(c, _k):
                        d = pl.ds(c * 16, 16)
                        labc = llab[d]
                        labhum = 1 - jnp.minimum(jnp.abs(labc - HUMAN), 1)
                        lkey[d] = jnp.where(labhum == 1, NEG, lkey[d])
                        return 0
                    lax.fori_loop(0, OWNC, ksweep, 0)

                @pl.when(capo)
                def _():
                    def ksweep(c, _k):
                        d = pl.ds(c * 16, 16)
                        labc = llab[d]
                        labhum = 1 - jnp.minimum(jnp.abs(labc - HUMAN), 1)
                        lkey[d] = jnp.where(labhum == 0, NEG, lkey[d])
                        return 0
                    lax.fori_loop(0, OWNC, ksweep, 0)

                def sweep(c, scarry):
                    sm1, si1, sm2, si2 = scarry
                    for k in range(4):
                        d = pl.ds(c * 64 + k * 16, 16)
                        idxv = base + c * 64 + k * 16 + lanes
                        a = lox1[d]
                        b = loy1[d]
                        cc = lox2[d]
                        dd = loy2[d]
                        areac = (cc - a) * (dd - b)
                        lt1 = jnp.maximum(ax1, a)
                        lt2 = jnp.maximum(ay1, b)
                        rb1 = jnp.minimum(ax2, cc)
                        rb2 = jnp.minimum(ay2, dd)
                        w1 = jnp.maximum(rb1 - lt1, 0.0)
                        hh1 = jnp.maximum(rb2 - lt2, 0.0)
                        in1 = w1 * hh1
                        un1 = aarea + areac - in1
                        iou1 = in1 / jnp.maximum(un1, np.float32(1e-8))
                        lt3 = jnp.maximum(sbx1, a)
                        lt4 = jnp.maximum(sby1, b)
                        rb3 = jnp.minimum(sbx2, cc)
                        rb4 = jnp.minimum(sby2, dd)
                        w2 = jnp.maximum(rb3 - lt3, 0.0)
                        hh2 = jnp.maximum(rb4 - lt4, 0.0)
                        in2 = w2 * hh2
                        un2 = sbarea + areac - in2
                        iou2 = in2 / jnp.maximum(un2, np.float32(1e-8))
                        kill = (iou1 > IOU_T) | (iou2 > IOU_T)
                        keyc = jnp.where(kill, NEG, lkey[d])
                        lkey[d] = keyc
                        upd = keyc > sm1
                        nm1 = jnp.where(upd, keyc, sm1)
                        ni1 = jnp.where(upd, idxv, si1)
                        dm = jnp.where(upd, sm1, keyc)
                        di = jnp.where(upd, si1, idxv)
                        upd2 = dm > sm2
                        sm1, si1 = nm1, ni1
                        sm2 = jnp.where(upd2, dm, sm2)
                        si2 = jnp.where(upd2, di, si2)
                    return sm1, si1, sm2, si2

                sm1, si1, sm2, si2 = lax.fori_loop(
                    0, OWNC // 4, sweep, zstate)
                sm1, si1, sm2, si2 = _argmax2_splat(sm1, si1, sm2, si2, lanes)
                rowbuf[pl.ds(0, 16)] = mk_row(sm1, si1, True)
                rowbuf[pl.ds(16, 16)] = mk_row(sm2, si2, True)

            hno[0] = h2
            hno[1] = o2
            hno[2] = ns + 1 + incb
            publish_round((t % 2) * PUB)

        return 0

    lax.fori_loop(0, NOUT, body, 0)

    @pl.when(is_out)
    def _():
        pltpu.sync_copy(pack_v, pack_o)
        pltpu.sync_copy(olab_v, olab_o)
        pltpu.sync_copy(oval_v, oval_o)


@jax.jit
def kernel(boxes, scores, labels):
    pad = NPAD - N
    x1 = jnp.pad(boxes[:, 0], (0, pad))
    y1 = jnp.pad(boxes[:, 1], (0, pad))
    x2 = jnp.pad(boxes[:, 2], (0, pad))
    y2 = jnp.pad(boxes[:, 3], (0, pad))
    sp = jnp.pad(scores, (0, pad))
    lp = jnp.pad(labels, (0, pad))

    mesh = plsc.VectorSubcoreMesh(core_axis_name="c", subcore_axis_name="s",
                                  num_cores=1)
    f = pl.kernel(
        _sc_nms, mesh=mesh,
        out_type=[
            jax.ShapeDtypeStruct((176,), jnp.float32),
            jax.ShapeDtypeStruct((48,), jnp.int32),
            jax.ShapeDtypeStruct((48,), jnp.int32),
        ],
        scratch_types=(
            [pltpu.VMEM((LPAD,), jnp.float32) for _ in range(5)]
            + [pltpu.VMEM((LPAD,), jnp.int32)]
            + [pltpu.VMEM((LPAD,), jnp.float32) for _ in range(6)]
            + [pltpu.VMEM((176,), jnp.float32),
               pltpu.VMEM((48,), jnp.int32),
               pltpu.VMEM((48,), jnp.int32),
               pltpu.VMEM((32,), jnp.float32),
               pltpu.VMEM((16,), jnp.float32),
               pltpu.SMEM((4,), jnp.int32),
               pltpu.VMEM((PUB,), jnp.float32),
               pltpu.VMEM_SHARED((3 * PUB,), jnp.float32)]),
    )
    pack, olab, oval = f(x1, y1, x2, y2, sp, lp)
    packed = pack[:150].reshape(30, 5)
    return packed, olab[:30], oval[:30].astype(bool)


# speedup vs baseline: 1.1411x; 1.1411x over previous
import jax
import jax.numpy as jnp
import numpy as np
from jax import lax
from jax.experimental import pallas as pl
from jax.experimental.pallas import tpu as pltpu
from jax.experimental.pallas import tpu_sc as plsc

N = 5000
NPAD = 5120
NW = 16
OWN = NPAD // NW
OWNC = OWN // 16
LPAD = OWN + 16
NEG = np.float32(-1e30)
NEG_HALF = np.float32(-5e29)
BLO = np.float32(-3e38)
SCORE_T = np.float32(0.2)
IOU_T = np.float32(0.5)
HUMAN = np.int32(1)
MAXH = np.int32(15)
MAXO = np.int32(15)
NOUT = 30

_GDN = lax.GatherDimensionNumbers(
    offset_dims=(), collapsed_slice_dims=(0,), start_index_map=(0,))


def _perm(x, idx):
    return lax.gather(x, idx[:, None], _GDN, (1,),
                      mode=lax.GatherScatterMode.PROMISE_IN_BOUNDS)


def _argmax_splat(m, mi, lanes):
    for k in (8, 4, 2, 1):
        pidx = lanes ^ k
        pm = _perm(m, pidx)
        pmi = _perm(mi, pidx)
        better = (pm > m) | ((pm == m) & (pmi < mi))
        m = jnp.where(better, pm, m)
        mi = jnp.where(better, pmi, mi)
    return m, mi


def _sc_nms(x1h, y1h, x2h, y2h, sh, labh, pack_o, olab_o, oval_o,
            lx1, ly1, lx2, ly2, ls, llab,
            lox1, loy1, lox2, loy2, lkey, lfill,
            pack_v, olab_v, oval_v, rowbuf, fbrow, puball, pub_sh):
    cid = lax.axis_index("c")
    sid = lax.axis_index("s")
    lanes = lax.iota(jnp.int32, 16)
    base = sid * OWN
    is_out = (cid == 0) & (sid == 0)

    pltpu.sync_copy(x1h.at[pl.ds(base, OWN)], lx1.at[pl.ds(0, OWN)])
    pltpu.sync_copy(y1h.at[pl.ds(base, OWN)], ly1.at[pl.ds(0, OWN)])
    pltpu.sync_copy(x2h.at[pl.ds(base, OWN)], lx2.at[pl.ds(0, OWN)])
    pltpu.sync_copy(y2h.at[pl.ds(base, OWN)], ly2.at[pl.ds(0, OWN)])
    pltpu.sync_copy(sh.at[pl.ds(base, OWN)], ls.at[pl.ds(0, OWN)])
    pltpu.sync_copy(labh.at[pl.ds(base, OWN)], llab.at[pl.ds(0, OWN)])

    def publish_round(off):
        pltpu.sync_copy(rowbuf, pub_sh.at[pl.ds(off + sid * 16, 16)])
        plsc.subcore_barrier()
        pltpu.sync_copy(pub_sh.at[pl.ds(off, NW * 16)], puball)

    def reduce_rows():
        bm = BLO
        bi = np.float32(3e38)
        brow = jnp.zeros((16,), jnp.float32)
        for r in range(NW):
            rrow = puball[pl.ds(r * 16, 16)]
            mr = rrow[0]
            ir = rrow[1]
            better = (mr > bm) | ((mr == bm) & (ir < bi))
            bf = jnp.where(better, np.float32(1.0), np.float32(0.0))
            bfv = jnp.full((16,), bf, jnp.float32)
            brow = bfv * rrow + (1.0 - bfv) * brow
            bm = jnp.where(better, mr, bm)
            bi = jnp.where(better, ir, bi)
        return brow

    def mk_row(m, mi, with_coords):
        ms = m[0]
        gi = mi[0]
        loff = jnp.clip(gi - base, 0, OWN - 1)
        fd = pl.ds(loff, 16)
        if with_coords:
            coords = (lox1[fd][0], loy1[fd][0], lox2[fd][0], loy2[fd][0])
        else:
            z = np.float32(0.0)
            coords = (z, z, z, z)
        vals = (ms, gi.astype(jnp.float32)) + coords + (
            lx1[fd][0], ly1[fd][0], lx2[fd][0], ly2[fd][0],
            ls[fd][0], llab[fd][0].astype(jnp.float32))
        row = jnp.full((16,), np.float32(0.0), jnp.float32)
        for j, v in enumerate(vals):
            row = jnp.where(lanes == j, v, row)
        return row

    def s1(c, m):
        d = pl.ds(c * 16, 16)
        return jnp.maximum(m, jnp.maximum(lx2[d], ly2[d]))
    mloc = lax.fori_loop(0, OWNC, s1, jnp.full((16,), NEG, jnp.float32))
    for k in (8, 4, 2, 1):
        mloc = jnp.maximum(mloc, _perm(mloc, lanes ^ k))
    rowbuf[...] = jnp.where(lanes == 0, mloc[0], np.float32(0.0))
    publish_round(2 * NW * 16)
    mc = BLO
    for r in range(NW):
        mc = jnp.maximum(mc, puball[pl.ds(r * 16, 16)][0])
    mc = mc + 1.0

    def s2(c, carry):
        m, mi = carry
        d = pl.ds(c * 16, 16)
        idxv = base + c * 16 + lanes
        off = llab[d].astype(jnp.float32) * mc
        a = lx1[d] + off
        b = ly1[d] + off
        cc = lx2[d] + off
        dd = ly2[d] + off
        lox1[d] = a
        loy1[d] = b
        lox2[d] = cc
        loy2[d] = dd
        sc = ls[d]
        vmask = sc >= SCORE_T
        keyc = jnp.where(vmask, sc, NEG)
        lkey[d] = keyc
        idxf = idxv.astype(jnp.float32)
        lfill[d] = jnp.where(vmask, sc,
                             jnp.where(idxv < N, -(idxf + 2.0), NEG))
        upd = keyc > m
        return jnp.where(upd, keyc, m), jnp.where(upd, idxv, mi)

    m0, mi0 = lax.fori_loop(
        0, OWNC, s2,
        (jnp.full((16,), NEG, jnp.float32), jnp.zeros((16,), jnp.int32)))
    m0, mi0 = _argmax_splat(m0, mi0, lanes)
    rowbuf[...] = mk_row(m0, mi0, True)
    fbrow[...] = jnp.zeros((16,), jnp.float32)
    publish_round(NW * 16)

    def body(t, carry):
        h, o = carry
        brow = reduce_rows()
        active = brow[0] > NEG_HALF

        @pl.when(jnp.logical_not(active))
        def _():
            def fsweep(c, fcarry):
                m, mi = fcarry
                fc = lfill[pl.ds(c * 16, 16)]
                idxv = base + c * 16 + lanes
                upd = fc > m
                return jnp.where(upd, fc, m), jnp.where(upd, idxv, mi)
            fm, fmi = lax.fori_loop(
                0, OWNC, fsweep,
                (jnp.full((16,), NEG, jnp.float32),
                 jnp.zeros((16,), jnp.int32)))
            fm, fmi = _argmax_splat(fm, fmi, lanes)
            rowbuf[...] = mk_row(fm, fmi, False)
            publish_round(2 * NW * 16)
            fbrow[...] = reduce_rows()

        af = jnp.where(active, np.float32(1.0), np.float32(0.0))
        afv = jnp.full((16,), af, jnp.float32)
        brow2 = afv * brow + (1.0 - afv) * fbrow[...]
        pick = brow2[1].astype(jnp.int32)
        plab = brow2[11].astype(jnp.int32)

        is_h = plab == HUMAN
        inc = active.astype(jnp.int32)
        h2 = h + jnp.where(is_h, inc, 0)
        o2 = o + jnp.where(is_h, 0, inc)

        loffp = jnp.clip(pick - base, 0, OWN - 1)
        inr = (pick >= base) & (pick < base + OWN)
        kd = pl.ds(loffp, 16)
        fold = lfill[kd]
        v0 = jnp.where(inr, NEG, fold[0])
        lfill[kd] = jnp.where(lanes == 0, v0, fold)

        @pl.when(is_out)
        def _():
            vals = jnp.where(lanes == 0, brow2[6],
                   jnp.where(lanes == 1, brow2[7],
                   jnp.where(lanes == 2, brow2[8],
                   jnp.where(lanes == 3, brow2[9], brow2[10]))))
            pd = pl.ds(t * 5, 16)
            pold = pack_v[pd]
            pack_v[pd] = jnp.where(lanes < 5, vals, pold)
            od = pl.ds(t, 16)
            lold = olab_v[od]
            olab_v[od] = jnp.where(lanes == 0, plab, lold)
            vold = oval_v[od]
            oval_v[od] = jnp.where(lanes == 0, inc, vold)

        rowbuf[...] = jnp.where(lanes == 0, NEG, np.float32(0.0))

        @pl.when(active)
        def _():
            pox1 = brow2[2]
            poy1 = brow2[3]
            pox2 = brow2[4]
            poy2 = brow2[5]
            poarea = (pox2 - pox1) * (poy2 - poy1)

            ph = 1 - jnp.minimum(jnp.abs(plab - HUMAN), 1)
            capchg = jnp.where(ph == 1, h2 == MAXH, o2 == MAXO)

            @pl.when(capchg)
            def _():
                phv = jnp.full((16,), ph, jnp.int32)

                def ksweep(c, _):
                    d = pl.ds(c * 16, 16)
                    labc = llab[d]
                    labhum = 1 - jnp.minimum(jnp.abs(labc - HUMAN), 1)
                    lkey[d] = jnp.where(labhum == phv, NEG, lkey[d])
                    return 0

                lax.fori_loop(0, OWNC, ksweep, 0)

            def sweep(c, scarry):
                m, mi = scarry
                for k in range(4):
                    d = pl.ds(c * 64 + k * 16, 16)
                    idxv = base + c * 64 + k * 16 + lanes
                    a = lox1[d]
                    b = loy1[d]
                    cc = lox2[d]
                    dd = loy2[d]
                    ltx = jnp.maximum(pox1, a)
                    lty = jnp.maximum(poy1, b)
                    rbx = jnp.minimum(pox2, cc)
                    rby = jnp.minimum(poy2, dd)
                    w = jnp.maximum(rbx - ltx, 0.0)
                    hh = jnp.maximum(rby - lty, 0.0)
                    inter = w * hh
                    union = poarea + (cc - a) * (dd - b) - inter
                    iou = inter / jnp.maximum(union, np.float32(1e-8))
                    keyc = jnp.where(iou > IOU_T, NEG, lkey[d])
                    lkey[d] = keyc
                    upd = keyc > m
                    m = jnp.where(upd, keyc, m)
                    mi = jnp.where(upd, idxv, mi)
                return m, mi

            m, mi = lax.fori_loop(
                0, OWNC // 4, sweep,
                (jnp.full((16,), NEG, jnp.float32),
                 jnp.zeros((16,), jnp.int32)))
            m, mi = _argmax_splat(m, mi, lanes)
            rowbuf[...] = mk_row(m, mi, True)

        publish_round((t % 2) * (NW * 16))
        return h2, o2

    lax.fori_loop(0, NOUT, body, (jnp.int32(0), jnp.int32(0)))

    @pl.when(is_out)
    def _():
        pltpu.sync_copy(pack_v, pack_o)
        pltpu.sync_copy(olab_v, olab_o)
        pltpu.sync_copy(oval_v, oval_o)


@jax.jit
def kernel(boxes, scores, labels):
    pad = NPAD - N
    x1 = jnp.pad(boxes[:, 0], (0, pad))
    y1 = jnp.pad(boxes[:, 1], (0, pad))
    x2 = jnp.pad(boxes[:, 2], (0, pad))
    y2 = jnp.pad(boxes[:, 3], (0, pad))
    sp = jnp.pad(scores, (0, pad))
    lp = jnp.pad(labels, (0, pad))

    mesh = plsc.VectorSubcoreMesh(core_axis_name="c", subcore_axis_name="s", num_cores=1)
    f = pl.kernel(
        _sc_nms, mesh=mesh,
        out_type=[
            jax.ShapeDtypeStruct((176,), jnp.float32),
            jax.ShapeDtypeStruct((48,), jnp.int32),
            jax.ShapeDtypeStruct((48,), jnp.int32),
        ],
        scratch_types=(
            [pltpu.VMEM((LPAD,), jnp.float32) for _ in range(5)]
            + [pltpu.VMEM((LPAD,), jnp.int32)]
            + [pltpu.VMEM((LPAD,), jnp.float32) for _ in range(6)]
            + [pltpu.VMEM((176,), jnp.float32),
               pltpu.VMEM((48,), jnp.int32),
               pltpu.VMEM((48,), jnp.int32),
               pltpu.VMEM((16,), jnp.float32),
               pltpu.VMEM((16,), jnp.float32),
               pltpu.VMEM((NW * 16,), jnp.float32),
               pltpu.VMEM_SHARED((3 * NW * 16,), jnp.float32)]),
    )
    pack, olab, oval = f(x1, y1, x2, y2, sp, lp)
    packed = pack[:150].reshape(30, 5)
    return packed, olab[:30], oval[:30].astype(bool)
